# Initial kernel scaffold; baseline (speedup 1.0000x reference)
import jax
import jax.numpy as jnp
from jax.experimental import pallas as pl
from jax.experimental.pallas import tpu as pltpu

_B = 16
_H = 512
_W = 512
_HW = _H * _W
_K = int(_HW * 0.1)
_TOTAL = _B * _HW
_CB = 2048
_NCHUNK = _HW // _CB


def _ohem_body(x_ref, t_ref, out_ref, cnt_ref, bits_ref):
    i = pl.program_id(0)

    @pl.when(i == 0)
    def _init():
        cnt_ref[0, 0] = 0.0

    @pl.when(i < _B)
    def _count_ones():
        cnt_ref[0, 0] += jnp.sum(t_ref[...])

    @pl.when(i >= _B)
    def _loss_row():
        row = i - _B
        cnt1 = cnt_ref[0, 0]
        cnt0 = jnp.float32(_TOTAL) - cnt1
        w0 = jnp.where(cnt0 == 0.0, jnp.float32(0.0), cnt1 / jnp.float32(_TOTAL))
        w1 = jnp.where(cnt1 == 0.0, jnp.float32(0.0), cnt0 / jnp.float32(_TOTAL))
        w0 = jnp.clip(w0, 0.2, 0.8)
        w1 = jnp.clip(w1, 0.2, 0.8)
        x = x_ref[...]
        t = t_ref[...]
        base = jnp.maximum(x, 0.0) - x * t + jnp.log1p(jnp.exp(-jnp.abs(x)))
        w = jnp.where(t == 0.0, w0, jnp.where(t == 1.0, w1, t))
        loss = base * w
        bits_ref[pl.ds(row, 1), :] = jax.lax.bitcast_convert_type(loss, jnp.int32)

    @pl.when(i == 2 * _B - 1)
    def _select():
        def count_ge(trial):
            def chunk(c, acc):
                blk = bits_ref[:, pl.ds(c * _CB, _CB)]
                return acc + (blk >= trial).astype(jnp.int32)
            acc = jax.lax.fori_loop(0, _NCHUNK, chunk,
                                    jnp.zeros((_B, _CB), jnp.int32))
            return jnp.sum(acc, axis=1, keepdims=True)

        def bit_step(j, cand):
            bit = 30 - j
            trial = cand | (jnp.int32(1) << bit)
            cnt = count_ge(trial)
            return jnp.where(cnt >= _K, trial, cand)

        thr = jax.lax.fori_loop(0, 31, bit_step, jnp.zeros((_B, 1), jnp.int32))

        def final_chunk(c, carry):
            cnt_acc, sum_acc = carry
            blk = bits_ref[:, pl.ds(c * _CB, _CB)]
            gt = blk > thr
            vals = jax.lax.bitcast_convert_type(blk, jnp.float32)
            cnt_acc = cnt_acc + gt.astype(jnp.int32)
            sum_acc = sum_acc + jnp.where(gt, vals, 0.0)
            return cnt_acc, sum_acc

        cnt_acc, sum_acc = jax.lax.fori_loop(
            0, _NCHUNK, final_chunk,
            (jnp.zeros((_B, _CB), jnp.int32), jnp.zeros((_B, _CB), jnp.float32)))
        cnt_gt = jnp.sum(cnt_acc, axis=1, keepdims=True)
        sum_gt = jnp.sum(sum_acc, axis=1, keepdims=True)
        thr_val = jax.lax.bitcast_convert_type(thr, jnp.float32)
        row_sum = sum_gt + (jnp.int32(_K) - cnt_gt).astype(jnp.float32) * thr_val
        out_ref[0, 0] = jnp.sum(row_sum) / jnp.float32(_B * _K)


def kernel(input, target):
    x = input.reshape(_B, _HW)
    t = target.reshape(_B, _HW)
    out = pl.pallas_call(
        _ohem_body,
        grid=(2 * _B,),
        in_specs=[
            pl.BlockSpec((1, _HW), lambda i: (jnp.maximum(i - _B, 0), 0)),
            pl.BlockSpec((1, _HW), lambda i: (i % _B, 0)),
        ],
        out_specs=pl.BlockSpec(memory_space=pltpu.SMEM),
        out_shape=jax.ShapeDtypeStruct((1, 1), jnp.float32),
        scratch_shapes=[
            pltpu.SMEM((1, 1), jnp.float32),
            pltpu.VMEM((_B, _HW), jnp.int32),
        ],
    )(x, t)
    return out[0, 0]


# speedup vs baseline: 12.9198x; 12.9198x over previous
import jax
import jax.numpy as jnp
from jax.experimental import pallas as pl
from jax.experimental.pallas import tpu as pltpu

_B = 16
_H = 512
_W = 512
_HW = _H * _W
_K = int(_HW * 0.1)
_TOTAL = _B * _HW
_CB = 2048
_NCHUNK = _HW // _CB


def _ohem_body(x_ref, t_ref, out_ref, cnt_ref, bits_ref):
    i = pl.program_id(0)

    @pl.when(i == 0)
    def _init():
        cnt_ref[0, 0] = 0.0

    @pl.when(i < _B)
    def _count_ones():
        cnt_ref[0, 0] += jnp.sum(t_ref[0])

    @pl.when(i >= _B)
    def _loss_row():
        row = i - _B
        cnt1 = cnt_ref[0, 0]
        cnt0 = jnp.float32(_TOTAL) - cnt1
        w0 = jnp.where(cnt0 == 0.0, jnp.float32(0.0), cnt1 / jnp.float32(_TOTAL))
        w1 = jnp.where(cnt1 == 0.0, jnp.float32(0.0), cnt0 / jnp.float32(_TOTAL))
        w0 = jnp.clip(w0, 0.2, 0.8)
        w1 = jnp.clip(w1, 0.2, 0.8)
        x = x_ref[0]
        t = t_ref[0]
        base = jnp.maximum(x, 0.0) - x * t + jnp.log1p(jnp.exp(-jnp.abs(x)))
        w = jnp.where(t == 0.0, w0, jnp.where(t == 1.0, w1, t))
        loss = base * w
        bits_ref[pl.ds(row, 1), :] = jax.lax.bitcast_convert_type(loss, jnp.int32)

    @pl.when(i == 2 * _B - 1)
    def _select():
        def count_ge(trial):
            def chunk(c, acc):
                blk = bits_ref[:, pl.ds(c * _CB, _CB)]
                return acc + (blk >= trial).astype(jnp.int32)
            acc = jax.lax.fori_loop(0, _NCHUNK, chunk,
                                    jnp.zeros((_B, _CB), jnp.int32))
            return jnp.sum(acc, axis=1, keepdims=True)

        def bit_step(j, cand):
            bit = 30 - j
            trial = cand | (jnp.int32(1) << bit)
            cnt = count_ge(trial)
            return jnp.where(cnt >= _K, trial, cand)

        thr = jax.lax.fori_loop(0, 31, bit_step, jnp.zeros((_B, 1), jnp.int32))

        def final_chunk(c, carry):
            cnt_acc, sum_acc = carry
            blk = bits_ref[:, pl.ds(c * _CB, _CB)]
            gt = blk > thr
            vals = jax.lax.bitcast_convert_type(blk, jnp.float32)
            cnt_acc = cnt_acc + gt.astype(jnp.int32)
            sum_acc = sum_acc + jnp.where(gt, vals, 0.0)
            return cnt_acc, sum_acc

        cnt_acc, sum_acc = jax.lax.fori_loop(
            0, _NCHUNK, final_chunk,
            (jnp.zeros((_B, _CB), jnp.int32), jnp.zeros((_B, _CB), jnp.float32)))
        cnt_gt = jnp.sum(cnt_acc, axis=1, keepdims=True)
        sum_gt = jnp.sum(sum_acc, axis=1, keepdims=True)
        thr_val = jax.lax.bitcast_convert_type(thr, jnp.float32)
        row_sum = sum_gt + (jnp.int32(_K) - cnt_gt).astype(jnp.float32) * thr_val
        out_ref[0, 0] = jnp.sum(row_sum) / jnp.float32(_B * _K)


def kernel(input, target):
    x = input.reshape(_B, 1, _HW)
    t = target.reshape(_B, 1, _HW)
    out = pl.pallas_call(
        _ohem_body,
        grid=(2 * _B,),
        in_specs=[
            pl.BlockSpec((1, 1, _HW), lambda i: (jnp.maximum(i - _B, 0), 0, 0)),
            pl.BlockSpec((1, 1, _HW), lambda i: (i % _B, 0, 0)),
        ],
        out_specs=pl.BlockSpec(memory_space=pltpu.SMEM),
        out_shape=jax.ShapeDtypeStruct((1, 1), jnp.float32),
        scratch_shapes=[
            pltpu.SMEM((1, 1), jnp.float32),
            pltpu.VMEM((_B, _HW), jnp.int32),
        ],
    )(x, t)
    return out[0, 0]


# speedup vs baseline: 16.4185x; 1.2708x over previous
import jax
import jax.numpy as jnp
from jax.experimental import pallas as pl
from jax.experimental.pallas import tpu as pltpu

_B = 16
_H = 512
_W = 512
_HW = _H * _W
_K = int(_HW * 0.1)
_TOTAL = _B * _HW
_LANE = 128
_SUB = _HW // _LANE
_CBS = 128
_NC = _SUB // _CBS
_CS = 32
_NCHUNK = _SUB // _CS


def _ohem_body(x_ref, t_ref, out_ref, cnt_ref, bits_ref):
    i = pl.program_id(0)

    @pl.when(i == 0)
    def _init():
        cnt_ref[0, 0] = 0.0

    @pl.when(i < _NC)
    def _count_ones():
        cnt_ref[0, 0] += jnp.sum(t_ref[...])

    @pl.when(i >= _NC)
    def _loss_chunk():
        c = i - _NC
        cnt1 = cnt_ref[0, 0]
        cnt0 = jnp.float32(_TOTAL) - cnt1
        w0 = jnp.where(cnt0 == 0.0, jnp.float32(0.0), cnt1 / jnp.float32(_TOTAL))
        w1 = jnp.where(cnt1 == 0.0, jnp.float32(0.0), cnt0 / jnp.float32(_TOTAL))
        w0 = jnp.clip(w0, 0.2, 0.8)
        w1 = jnp.clip(w1, 0.2, 0.8)
        x = x_ref[...]
        t = t_ref[...]
        base = jnp.maximum(x, 0.0) - x * t + jnp.log1p(jnp.exp(-jnp.abs(x)))
        w = jnp.where(t == 0.0, w0, jnp.where(t == 1.0, w1, t))
        loss = base * w
        bits_ref[:, pl.ds(c * _CBS, _CBS), :] = (
            jax.lax.bitcast_convert_type(loss, jnp.int32))

    @pl.when(i == 2 * _NC - 1)
    def _select():
        def count_ge(trial):
            def chunk(c, acc):
                blk = bits_ref[:, pl.ds(c * _CS, _CS), :]
                return acc + (blk >= trial).astype(jnp.int32)
            acc = jax.lax.fori_loop(0, _NCHUNK, chunk,
                                    jnp.zeros((_B, _CS, _LANE), jnp.int32))
            return jnp.sum(acc, axis=(1, 2), keepdims=True)

        def bit_step(j, cand):
            bit = 30 - j
            trial = cand | (jnp.int32(1) << bit)
            cnt = count_ge(trial)
            return jnp.where(cnt >= _K, trial, cand)

        thr = jax.lax.fori_loop(0, 31, bit_step,
                                jnp.zeros((_B, 1, 1), jnp.int32))

        def final_chunk(c, carry):
            cnt_acc, sum_acc = carry
            blk = bits_ref[:, pl.ds(c * _CS, _CS), :]
            gt = blk > thr
            vals = jax.lax.bitcast_convert_type(blk, jnp.float32)
            cnt_acc = cnt_acc + gt.astype(jnp.int32)
            sum_acc = sum_acc + jnp.where(gt, vals, 0.0)
            return cnt_acc, sum_acc

        cnt_acc, sum_acc = jax.lax.fori_loop(
            0, _NCHUNK, final_chunk,
            (jnp.zeros((_B, _CS, _LANE), jnp.int32),
             jnp.zeros((_B, _CS, _LANE), jnp.float32)))
        cnt_gt = jnp.sum(cnt_acc, axis=(1, 2), keepdims=True)
        sum_gt = jnp.sum(sum_acc, axis=(1, 2), keepdims=True)
        thr_val = jax.lax.bitcast_convert_type(thr, jnp.float32)
        row_sum = sum_gt + (jnp.int32(_K) - cnt_gt).astype(jnp.float32) * thr_val
        out_ref[0, 0] = jnp.sum(row_sum) / jnp.float32(_B * _K)


def kernel(input, target):
    x = input.reshape(_B, _SUB, _LANE)
    t = target.reshape(_B, _SUB, _LANE)
    out = pl.pallas_call(
        _ohem_body,
        grid=(2 * _NC,),
        in_specs=[
            pl.BlockSpec((_B, _CBS, _LANE),
                         lambda i: (0, jnp.maximum(i - _NC, 0), 0)),
            pl.BlockSpec((_B, _CBS, _LANE), lambda i: (0, i % _NC, 0)),
        ],
        out_specs=pl.BlockSpec(memory_space=pltpu.SMEM),
        out_shape=jax.ShapeDtypeStruct((1, 1), jnp.float32),
        scratch_shapes=[
            pltpu.SMEM((1, 1), jnp.float32),
            pltpu.VMEM((_B, _SUB, _LANE), jnp.int32),
        ],
    )(x, t)
    return out[0, 0]
